# baseline (speedup 1.0000x reference)
import functools

import jax
import jax.numpy as jnp
from jax import lax
from jax.experimental import pallas as pl
from jax.experimental.pallas import tpu as pltpu
from jax.experimental.pallas import tpu_sc as plsc


def _table_body(emb_ref, w_ref, b_ref, g_ref, beta_ref, out_ref):
    h = lax.dot_general(
        emb_ref[...], w_ref[...], (((1,), (1,)), ((), ())),
        preferred_element_type=jnp.float32,
    )
    h = h + b_ref[...]
    mu = jnp.mean(h, axis=-1, keepdims=True)
    d = h - mu
    var = jnp.mean(d * d, axis=-1, keepdims=True)
    t = (d * lax.rsqrt(var + 1e-5)) * g_ref[...] + beta_ref[...]
    V, D = t.shape
    left = jnp.broadcast_to(t[:, None, :], (V, V, D))
    right = jnp.broadcast_to(t[None, :, :], (V, V, D))
    out_ref[...] = jnp.concatenate([left, right], axis=-1)


def _make_paired_table(emb, W, b, gamma, beta):
    V, D = emb.shape
    paired = pl.pallas_call(
        _table_body,
        out_shape=jax.ShapeDtypeStruct((V, V, 2 * D), jnp.float32),
    )(emb, W, b.reshape(1, D), gamma.reshape(1, D), beta.reshape(1, D))
    return paired.reshape(V * V, 2 * D)


_NSLOT = 4


def _make_gather(N2, n_workers, chunk):
    n_per_w = N2 // n_workers
    n_chunks = n_per_w // chunk
    n_outer = n_chunks // _NSLOT
    mesh = plsc.VectorSubcoreMesh(core_axis_name="c", subcore_axis_name="s")

    scratch = (
        [pltpu.VMEM((n_per_w,), jnp.int32)]
        + [pltpu.VMEM((chunk, 128), jnp.float32) for _ in range(_NSLOT)]
        + [pltpu.SemaphoreType.DMA for _ in range(2 * _NSLOT + 1)]
    )

    @functools.partial(
        pl.kernel,
        out_type=jax.ShapeDtypeStruct((N2, 128), jnp.float32),
        mesh=mesh,
        scratch_types=scratch,
    )
    def gather_k(tab_hbm, idx_hbm, out_hbm, *refs):
        idx_v = refs[0]
        rows = refs[1:1 + _NSLOT]
        sem_g = refs[1 + _NSLOT:1 + 2 * _NSLOT]
        sem_s = refs[1 + 2 * _NSLOT:1 + 3 * _NSLOT]
        sem_ld = refs[1 + 3 * _NSLOT]

        wid = lax.axis_index("s") * 2 + lax.axis_index("c")
        base = pl.multiple_of(wid * n_per_w, n_per_w)

        pltpu.async_copy(idx_hbm.at[pl.ds(base, n_per_w)], idx_v, sem_ld)
        pltpu.make_async_copy(
            idx_hbm.at[pl.ds(base, n_per_w)], idx_v, sem_ld
        ).wait()

        def fire_gather(g, b):
            pltpu.async_copy(
                tab_hbm.at[idx_v.at[pl.ds(g * chunk, chunk)]],
                rows[b],
                sem_g[b],
            )

        def wait_gather(b):
            pltpu.make_async_copy(
                tab_hbm.at[idx_v.at[pl.ds(0, chunk)]], rows[b], sem_g[b]
            ).wait()

        def fire_store(g, b):
            pltpu.async_copy(
                rows[b], out_hbm.at[pl.ds(base + g * chunk, chunk)], sem_s[b]
            )

        def wait_store(b):
            pltpu.make_async_copy(
                rows[b], out_hbm.at[pl.ds(base, chunk)], sem_s[b]
            ).wait()

        def outer(i, carry):
            g0 = i * _NSLOT
            for b in range(_NSLOT):
                g = g0 + b
                bprev = (b - 1) % _NSLOT

                @pl.when(i > 0)
                def _():
                    wait_store(b)

                fire_gather(g, b)

                if b > 0:
                    wait_gather(bprev)
                    fire_store(g - 1, bprev)
                else:
                    @pl.when(i > 0)
                    def _():
                        wait_gather(bprev)
                        fire_store(g - 1, bprev)

            return carry

        lax.fori_loop(0, n_outer, outer, 0)

        last = _NSLOT - 1
        wait_gather(last)
        fire_store(n_chunks - 1, last)
        for b in range(_NSLOT):
            wait_store(b)

    return gather_k


def kernel(x, emb, W, b, gamma, beta):
    B, L = x.shape
    V, D = emb.shape
    N2 = (B * L) // 2
    paired = _make_paired_table(emb, W, b, gamma, beta)
    xf = x.astype(jnp.int32).reshape(N2, 2)
    idx2 = xf[:, 0] * V + xf[:, 1]
    gather = _make_gather(N2, n_workers=32, chunk=128)
    out2 = gather(paired, idx2)
    return out2.reshape(B, L, D)
